# baseline (speedup 1.0000x reference)
import functools

import jax
import jax.numpy as jnp
from jax import lax
from jax.experimental import pallas as pl
from jax.experimental.pallas import tpu as pltpu
from jax.experimental.pallas import tpu_sc as plsc

N_NODES = 10000
D = 128
N_EDGES = 320000
N_REL = 8
N_BASES = 4

NC = 2
NS = 16
LANES = 16
NW = NC * NS
EDGES_PER_W = N_EDGES // NW
CHUNK = 128
FULL_CHUNKS = EDGES_PER_W // CHUNK
TAIL = EDGES_PER_W - FULL_CHUNKS * CHUNK
ROWS_PER_SUB = 624
ROWS_TAIL = N_NODES - NS * ROWS_PER_SUB


_IDX_ROWS = N_EDGES // D
_IDX_BLK = 320


def _xw_body(att_ref, x_ref, basis_ref, et_ref, src_ref, root_ref, bias_ref,
             out_ref, idx_ref, self_ref):
    r = pl.program_id(0)
    w = (att_ref[r, 0] * basis_ref[0] + att_ref[r, 1] * basis_ref[1]
         + att_ref[r, 2] * basis_ref[2] + att_ref[r, 3] * basis_ref[3])
    out_ref[...] = jnp.dot(x_ref[...], w, preferred_element_type=jnp.float32)
    idx_ref[...] = et_ref[...] * N_NODES + src_ref[...]

    @pl.when(r == 0)
    def _selfloop():
        self_ref[...] = (
            jnp.dot(x_ref[...], root_ref[...],
                    preferred_element_type=jnp.float32) + bias_ref[...])


def _xw_table(att, x, basis, et2d, src2d, root, bias2d):
    return pl.pallas_call(
        _xw_body,
        grid=(N_REL,),
        in_specs=[
            pl.BlockSpec(memory_space=pltpu.SMEM),
            pl.BlockSpec((N_NODES, D), lambda r: (0, 0)),
            pl.BlockSpec((N_BASES, D, D), lambda r: (0, 0, 0)),
            pl.BlockSpec((_IDX_BLK, D), lambda r: (r, 0)),
            pl.BlockSpec((_IDX_BLK, D), lambda r: (r, 0)),
            pl.BlockSpec((D, D), lambda r: (0, 0)),
            pl.BlockSpec((1, D), lambda r: (0, 0)),
        ],
        out_specs=[
            pl.BlockSpec((N_NODES, D), lambda r: (r, 0)),
            pl.BlockSpec((_IDX_BLK, D), lambda r: (r, 0)),
            pl.BlockSpec((N_NODES, D), lambda r: (0, 0)),
        ],
        out_shape=[
            jax.ShapeDtypeStruct((N_REL * N_NODES, D), jnp.float32),
            jax.ShapeDtypeStruct((_IDX_ROWS, D), jnp.int32),
            jax.ShapeDtypeStruct((N_NODES, D), jnp.float32),
        ],
    )(att, x, basis, et2d, src2d, root, bias2d)


_MESH = plsc.VectorSubcoreMesh(core_axis_name="c", subcore_axis_name="s")


@functools.partial(
    pl.kernel,
    mesh=_MESH,
    out_type=jax.ShapeDtypeStruct((NC * N_NODES, D), jnp.float32),
    scratch_types=[
        pltpu.VMEM((EDGES_PER_W,), jnp.int32),
        pltpu.VMEM((CHUNK,), jnp.int32),
        pltpu.VMEM((CHUNK,), jnp.int32),
        pltpu.VMEM((CHUNK, D), jnp.float32),
        pltpu.VMEM((CHUNK, D), jnp.float32),
        pltpu.VMEM((TAIL,), jnp.int32),
        pltpu.VMEM((TAIL, D), jnp.float32),
        pltpu.VMEM_SHARED((N_NODES, D), jnp.float32),
        pltpu.SemaphoreType.DMA,
        pltpu.SemaphoreType.DMA,
        pltpu.SemaphoreType.DMA,
        pltpu.SemaphoreType.DMA,
    ],
)
def _sc_aggregate(xw_hbm, idx_hbm, dst_hbm, zeros_hbm, self_hbm, out_hbm,
                  idx_all, dstg0, dstg1, rows0, rows1,
                  dstgt, rows_t, acc_sh,
                  gsem0, gsem1, dsem0, dsem1):
    c = lax.axis_index("c")
    s = lax.axis_index("s")
    wid = c * NS + s
    base_w = wid * EDGES_PER_W

    dstg = (dstg0, dstg1)
    rows = (rows0, rows1)
    gsem = (gsem0, gsem1)
    dsem = (dsem0, dsem1)

    h_idx = pltpu.async_copy(idx_hbm.at[pl.ds(base_w, EDGES_PER_W)],
                             idx_all, gsem0)

    row0 = s * ROWS_PER_SUB

    @pl.when(c == 0)
    def _init_self():
        pltpu.sync_copy(self_hbm.at[pl.ds(row0, ROWS_PER_SUB)],
                        acc_sh.at[pl.ds(row0, ROWS_PER_SUB)])

        @pl.when(s == NS - 1)
        def _init_self_tail():
            pltpu.sync_copy(self_hbm.at[pl.ds(NS * ROWS_PER_SUB, ROWS_TAIL)],
                            acc_sh.at[pl.ds(NS * ROWS_PER_SUB, ROWS_TAIL)])

    @pl.when(c == 1)
    def _init_zero():
        pltpu.sync_copy(zeros_hbm,
                        acc_sh.at[pl.ds(row0, ROWS_PER_SUB)])

        @pl.when(s == NS - 1)
        def _init_zero_tail():
            pltpu.sync_copy(zeros_hbm.at[pl.ds(0, ROWS_TAIL)],
                            acc_sh.at[pl.ds(NS * ROWS_PER_SUB, ROWS_TAIL)])

    h_idx.wait()

    def fire(g, k):
        pltpu.async_copy(xw_hbm.at[idx_all.at[pl.ds(g * CHUNK, CHUNK)]],
                         rows[k], gsem[k])
        pltpu.async_copy(dst_hbm.at[pl.ds(base_w + g * CHUNK, CHUNK)],
                         dstg[k], dsem[k])

    def drain(g, k):
        pltpu.make_async_copy(xw_hbm.at[idx_all.at[pl.ds(g * CHUNK, CHUNK)]],
                              rows[k], gsem[k]).wait()
        pltpu.make_async_copy(dst_hbm.at[pl.ds(base_w + g * CHUNK, CHUNK)],
                              dstg[k], dsem[k]).wait()

    fire(0, 0)
    plsc.subcore_barrier()

    def pipe_body(i, carry):
        g0 = 2 * i
        fire(g0 + 1, 1)
        drain(g0, 0)
        pltpu.sync_copy(rows0, acc_sh.at[dstg0], add=True)

        @pl.when(i < FULL_CHUNKS // 2 - 1)
        def _refire():
            fire(g0 + 2, 0)

        drain(g0 + 1, 1)
        pltpu.sync_copy(rows1, acc_sh.at[dstg1], add=True)
        return carry

    lax.fori_loop(0, FULL_CHUNKS // 2, pipe_body, 0)

    t0 = FULL_CHUNKS * CHUNK
    pltpu.async_copy(dst_hbm.at[pl.ds(base_w + t0, TAIL)], dstgt,
                     dsem0).wait()
    pltpu.async_copy(xw_hbm.at[idx_all.at[pl.ds(t0, TAIL)]],
                     rows_t, gsem0).wait()
    pltpu.sync_copy(rows_t, acc_sh.at[dstgt], add=True)

    plsc.subcore_barrier()
    row0 = s * ROWS_PER_SUB
    pltpu.sync_copy(acc_sh.at[pl.ds(row0, ROWS_PER_SUB)],
                    out_hbm.at[pl.ds(c * N_NODES + row0, ROWS_PER_SUB)])

    @pl.when(s == NS - 1)
    def _write_tail():
        t0 = NS * ROWS_PER_SUB
        pltpu.sync_copy(acc_sh.at[pl.ds(t0, ROWS_TAIL)],
                        out_hbm.at[pl.ds(c * N_NODES + t0, ROWS_TAIL)])


_CBLK = 10000


def _combine_body(p0_ref, p1_ref, out_ref):
    out_ref[...] = p0_ref[...] + p1_ref[...]


def _combine(partials):
    nblk = N_NODES // _CBLK
    return pl.pallas_call(
        _combine_body,
        grid=(nblk,),
        in_specs=[
            pl.BlockSpec((_CBLK, D), lambda i: (i, 0)),
            pl.BlockSpec((_CBLK, D), lambda i, _n=nblk: (i + _n, 0)),
        ],
        out_specs=pl.BlockSpec((_CBLK, D), lambda i: (i, 0)),
        out_shape=jax.ShapeDtypeStruct((N_NODES, D), jnp.float32),
    )(partials, partials)


def kernel(x, edge_index, edge_type, basis, att, root, bias):
    src = edge_index[0].astype(jnp.int32)
    dst = edge_index[1].astype(jnp.int32)
    et = edge_type.astype(jnp.int32)
    xw, idx2d, selfloop = _xw_table(att, x, basis,
                                    et.reshape(-1, D), src.reshape(-1, D),
                                    root, bias.reshape(1, D))
    zeros = jnp.zeros((ROWS_PER_SUB, D), jnp.float32)
    partials = _sc_aggregate(xw, idx2d.reshape(-1), dst, zeros, selfloop)
    return _combine(partials)


# speedup vs baseline: 1.0062x; 1.0062x over previous
import functools

import jax
import jax.numpy as jnp
from jax import lax
from jax.experimental import pallas as pl
from jax.experimental.pallas import tpu as pltpu
from jax.experimental.pallas import tpu_sc as plsc

N_NODES = 10000
D = 128
N_EDGES = 320000
N_REL = 8
N_BASES = 4

NC = 2
NS = 16
LANES = 16
NW = NC * NS
EDGES_PER_W = N_EDGES // NW
CHUNK = 128
FULL_CHUNKS = EDGES_PER_W // CHUNK
TAIL = EDGES_PER_W - FULL_CHUNKS * CHUNK
ROWS_PER_SUB = 624
ROWS_TAIL = N_NODES - NS * ROWS_PER_SUB


_IDX_ROWS = N_EDGES // D
_IDX_BLK = 320


def _xw_body(att_ref, x_ref, basis_ref, et_ref, src_ref, root_ref, bias_ref,
             out_ref, idx_ref, self_ref):
    r = pl.program_id(0)
    w = (att_ref[r, 0] * basis_ref[0] + att_ref[r, 1] * basis_ref[1]
         + att_ref[r, 2] * basis_ref[2] + att_ref[r, 3] * basis_ref[3])
    out_ref[...] = jnp.dot(x_ref[...], w, preferred_element_type=jnp.float32)
    idx_ref[...] = et_ref[...] * N_NODES + src_ref[...]

    @pl.when(r == 0)
    def _selfloop():
        self_ref[...] = (
            jnp.dot(x_ref[...], root_ref[...],
                    preferred_element_type=jnp.float32) + bias_ref[...])


def _xw_table(att, x, basis, et2d, src2d, root, bias2d):
    return pl.pallas_call(
        _xw_body,
        grid=(N_REL,),
        in_specs=[
            pl.BlockSpec(memory_space=pltpu.SMEM),
            pl.BlockSpec((N_NODES, D), lambda r: (0, 0)),
            pl.BlockSpec((N_BASES, D, D), lambda r: (0, 0, 0)),
            pl.BlockSpec((_IDX_BLK, D), lambda r: (r, 0)),
            pl.BlockSpec((_IDX_BLK, D), lambda r: (r, 0)),
            pl.BlockSpec((D, D), lambda r: (0, 0)),
            pl.BlockSpec((1, D), lambda r: (0, 0)),
        ],
        out_specs=[
            pl.BlockSpec((N_NODES, D), lambda r: (r, 0)),
            pl.BlockSpec((_IDX_BLK, D), lambda r: (r, 0)),
            pl.BlockSpec((N_NODES, D), lambda r: (0, 0)),
        ],
        out_shape=[
            jax.ShapeDtypeStruct((N_REL * N_NODES, D), jnp.float32),
            jax.ShapeDtypeStruct((_IDX_ROWS, D), jnp.int32),
            jax.ShapeDtypeStruct((N_NODES, D), jnp.float32),
        ],
    )(att, x, basis, et2d, src2d, root, bias2d)


_MESH = plsc.VectorSubcoreMesh(core_axis_name="c", subcore_axis_name="s")


@functools.partial(
    pl.kernel,
    mesh=_MESH,
    out_type=jax.ShapeDtypeStruct((NC * N_NODES, D), jnp.float32),
    scratch_types=[
        pltpu.VMEM((EDGES_PER_W,), jnp.int32),
        pltpu.VMEM((CHUNK,), jnp.int32),
        pltpu.VMEM((CHUNK,), jnp.int32),
        pltpu.VMEM((CHUNK, D), jnp.float32),
        pltpu.VMEM((CHUNK, D), jnp.float32),
        pltpu.VMEM((TAIL,), jnp.int32),
        pltpu.VMEM((TAIL, D), jnp.float32),
        pltpu.VMEM_SHARED((N_NODES, D), jnp.float32),
        pltpu.SemaphoreType.DMA,
        pltpu.SemaphoreType.DMA,
        pltpu.SemaphoreType.DMA,
        pltpu.SemaphoreType.DMA,
    ],
)
def _sc_aggregate(xw_hbm, idx_hbm, dst_hbm, zeros_hbm, self_hbm, out_hbm,
                  idx_all, dstg0, dstg1, rows0, rows1,
                  dstgt, rows_t, acc_sh,
                  gsem0, gsem1, dsem0, dsem1):
    c = lax.axis_index("c")
    s = lax.axis_index("s")
    wid = c * NS + s
    base_w = wid * EDGES_PER_W

    dstg = (dstg0, dstg1)
    rows = (rows0, rows1)
    gsem = (gsem0, gsem1)
    dsem = (dsem0, dsem1)

    h_idx = pltpu.async_copy(idx_hbm.at[pl.ds(base_w, EDGES_PER_W)],
                             idx_all, gsem0)

    row0 = s * ROWS_PER_SUB

    @pl.when(c == 0)
    def _init_self():
        pltpu.sync_copy(self_hbm.at[pl.ds(row0, ROWS_PER_SUB)],
                        acc_sh.at[pl.ds(row0, ROWS_PER_SUB)])

        @pl.when(s == NS - 1)
        def _init_self_tail():
            pltpu.sync_copy(self_hbm.at[pl.ds(NS * ROWS_PER_SUB, ROWS_TAIL)],
                            acc_sh.at[pl.ds(NS * ROWS_PER_SUB, ROWS_TAIL)])

    @pl.when(c == 1)
    def _init_zero():
        pltpu.sync_copy(zeros_hbm,
                        acc_sh.at[pl.ds(row0, ROWS_PER_SUB)])

        @pl.when(s == NS - 1)
        def _init_zero_tail():
            pltpu.sync_copy(zeros_hbm.at[pl.ds(0, ROWS_TAIL)],
                            acc_sh.at[pl.ds(NS * ROWS_PER_SUB, ROWS_TAIL)])

    h_idx.wait()

    def fire(g, k):
        pltpu.async_copy(xw_hbm.at[idx_all.at[pl.ds(g * CHUNK, CHUNK)]],
                         rows[k], gsem[k])
        pltpu.async_copy(dst_hbm.at[pl.ds(base_w + g * CHUNK, CHUNK)],
                         dstg[k], dsem[k])

    def drain(g, k):
        pltpu.make_async_copy(xw_hbm.at[idx_all.at[pl.ds(g * CHUNK, CHUNK)]],
                              rows[k], gsem[k]).wait()
        pltpu.make_async_copy(dst_hbm.at[pl.ds(base_w + g * CHUNK, CHUNK)],
                              dstg[k], dsem[k]).wait()

    fire(0, 0)
    plsc.subcore_barrier()

    def pipe_body(i, carry):
        g0 = 2 * i
        fire(g0 + 1, 1)
        drain(g0, 0)
        pltpu.sync_copy(rows0, acc_sh.at[dstg0], add=True)

        @pl.when(i < FULL_CHUNKS // 2 - 1)
        def _refire():
            fire(g0 + 2, 0)

        drain(g0 + 1, 1)
        pltpu.sync_copy(rows1, acc_sh.at[dstg1], add=True)
        return carry

    lax.fori_loop(0, FULL_CHUNKS // 2, pipe_body, 0)

    t0 = FULL_CHUNKS * CHUNK
    pltpu.async_copy(dst_hbm.at[pl.ds(base_w + t0, TAIL)], dstgt,
                     dsem0).wait()
    pltpu.async_copy(xw_hbm.at[idx_all.at[pl.ds(t0, TAIL)]],
                     rows_t, gsem0).wait()
    pltpu.sync_copy(rows_t, acc_sh.at[dstgt], add=True)

    plsc.subcore_barrier()
    row0 = s * ROWS_PER_SUB
    pltpu.sync_copy(acc_sh.at[pl.ds(row0, ROWS_PER_SUB)],
                    out_hbm.at[pl.ds(c * N_NODES + row0, ROWS_PER_SUB)])

    @pl.when(s == NS - 1)
    def _write_tail():
        t0 = NS * ROWS_PER_SUB
        pltpu.sync_copy(acc_sh.at[pl.ds(t0, ROWS_TAIL)],
                        out_hbm.at[pl.ds(c * N_NODES + t0, ROWS_TAIL)])


_CBLK = 5000


def _combine_body(p0_ref, p1_ref, out_ref):
    out_ref[...] = p0_ref[...] + p1_ref[...]


def _combine(partials):
    nblk = N_NODES // _CBLK
    return pl.pallas_call(
        _combine_body,
        grid=(nblk,),
        in_specs=[
            pl.BlockSpec((_CBLK, D), lambda i: (i, 0)),
            pl.BlockSpec((_CBLK, D), lambda i, _n=nblk: (i + _n, 0)),
        ],
        out_specs=pl.BlockSpec((_CBLK, D), lambda i: (i, 0)),
        out_shape=jax.ShapeDtypeStruct((N_NODES, D), jnp.float32),
    )(partials, partials)


def kernel(x, edge_index, edge_type, basis, att, root, bias):
    src = edge_index[0].astype(jnp.int32)
    dst = edge_index[1].astype(jnp.int32)
    et = edge_type.astype(jnp.int32)
    xw, idx2d, selfloop = _xw_table(att, x, basis,
                                    et.reshape(-1, D), src.reshape(-1, D),
                                    root, bias.reshape(1, D))
    zeros = jnp.zeros((ROWS_PER_SUB, D), jnp.float32)
    partials = _sc_aggregate(xw, idx2d.reshape(-1), dst, zeros, selfloop)
    return _combine(partials)
